# Initial kernel scaffold; baseline (speedup 1.0000x reference)
import jax
import jax.numpy as jnp
from jax.experimental import pallas as pl


def kernel(rel, table, indices):
    raise NotImplementedError("write your pallas kernel here")


# speedup vs baseline: 1.0254x; 1.0254x over previous
import jax
import jax.numpy as jnp
from jax.experimental import pallas as pl
from jax.experimental.pallas import tpu as pltpu

_BLOCK_B = 2048


def _bcast_kernel(idx_ref, table_ref, o0_ref, o1_ref, rows_vmem, sem):
    i = pl.program_id(0)

    @pl.when(i == 0)
    def _fetch_rows():
        cp0 = pltpu.make_async_copy(
            table_ref.at[pl.ds(idx_ref[0], 1), :], rows_vmem.at[0:1, :], sem
        )
        cp0.start()
        cp0.wait()
        cp1 = pltpu.make_async_copy(
            table_ref.at[pl.ds(idx_ref[1], 1), :], rows_vmem.at[1:2, :], sem
        )
        cp1.start()
        cp1.wait()

    o0_ref[...] = jnp.broadcast_to(rows_vmem[0, :], o0_ref.shape)
    o1_ref[...] = jnp.broadcast_to(rows_vmem[1, :], o1_ref.shape)


def kernel(rel, table, indices):
    batch = rel.shape[0]
    d = table.shape[1]
    block_b = min(_BLOCK_B, batch)
    grid = (batch // block_b,)
    grid_spec = pltpu.PrefetchScalarGridSpec(
        num_scalar_prefetch=1,
        grid=grid,
        in_specs=[pl.BlockSpec(memory_space=pl.ANY)],
        out_specs=[
            pl.BlockSpec((block_b, d), lambda i, idx: (i, 0)),
            pl.BlockSpec((block_b, d), lambda i, idx: (i, 0)),
        ],
        scratch_shapes=[
            pltpu.VMEM((2, d), jnp.float32),
            pltpu.SemaphoreType.DMA,
        ],
    )
    o0, o1 = pl.pallas_call(
        _bcast_kernel,
        grid_spec=grid_spec,
        out_shape=[
            jax.ShapeDtypeStruct((batch, d), jnp.float32),
            jax.ShapeDtypeStruct((batch, d), jnp.float32),
        ],
    )(indices, table)
    return (o0, o1)
